# baseline (speedup 1.0000x reference)
import functools

import jax
import jax.numpy as jnp
from jax import lax
from jax.experimental import pallas as pl
from jax.experimental.pallas import tpu as pltpu
from jax.experimental.pallas import tpu_sc as plsc

N, D = 16384, 512
H, C = 128, 32
K = 1024
BN = 512
NB = N // BN
EPS = 1e-5
INV_N = 1.0 / N

NW = 32
RPW = N // NW
CHUNK = 128
NCH = RPW // CHUNK


def _mm(a, b):
    return jnp.dot(a.astype(jnp.bfloat16), b.astype(jnp.bfloat16),
                   preferred_element_type=jnp.float32)


def _enc_body(X_ref, W1_ref, b1_ref, g1_ref, be1_ref, W2_ref, b2_ref,
              dW1_ref, db1_ref, cb_ref, cbT_ref, b2row_ref,
              topics_ref, zl_ref, M_ref,
              h1_scr, s1, v1):
    p = pl.program_id(0)
    b = pl.program_id(1)
    rows = pl.ds(b * BN, BN)

    @pl.when(p == 0)
    def _p0():
        h = _mm(X_ref[...], W1_ref[...]) + b1_ref[...]
        h1_scr[rows, :] = h
        blk = jnp.sum(h, axis=0, keepdims=True)

        @pl.when(b == 0)
        def _():
            s1[...] = blk
            M_ref[...] = _mm(cb_ref[...], dW1_ref[...]) + db1_ref[...]

        @pl.when(b != 0)
        def _():
            s1[...] += blk

    @pl.when(p == 1)
    def _p1():
        mu = s1[...] * INV_N
        d = h1_scr[rows, :] - mu
        blk = jnp.sum(d * d, axis=0, keepdims=True)

        @pl.when(b == 0)
        def _():
            v1[...] = blk

        @pl.when(b != 0)
        def _():
            v1[...] += blk

    @pl.when(p == 2)
    def _p2():
        mu = s1[...] * INV_N
        sd = jnp.sqrt(v1[...] * INV_N + EPS)
        t = (h1_scr[rows, :] - mu) / sd * g1_ref[...] + be1_ref[...]
        r = jnp.maximum(t, 0.0)
        z = _mm(r, W2_ref[...]) + b2_ref[...]
        a2 = jnp.sum(z * z, axis=1, keepdims=True)
        ab = _mm(z, cbT_ref[...])
        dist = (a2 - 2.0 * ab) + b2row_ref[...]
        mn = jnp.min(dist, axis=1, keepdims=True)
        iota = lax.broadcasted_iota(jnp.int32, (BN, K), 1)
        am = jnp.min(jnp.where(dist == mn, iota, K), axis=1, keepdims=True)
        topics_ref[rows, :] = am
        blk_zl = jnp.sum(mn, axis=0, keepdims=True)

        @pl.when(b == 0)
        def _():
            zl_ref[...] = blk_zl

        @pl.when(b != 0)
        def _():
            zl_ref[...] += blk_zl


def _x_once(phase):
    def f(p, b):
        return (lax.select(p == phase, b, 0), 0)
    return f


def _const(p, b):
    return (0, 0)


_sc_mesh = plsc.VectorSubcoreMesh(core_axis_name="c", subcore_axis_name="s")


@functools.partial(
    pl.kernel, mesh=_sc_mesh,
    out_type=jax.ShapeDtypeStruct((N, H), jnp.float32),
    scratch_types=[
        pltpu.VMEM((NCH, CHUNK), jnp.int32),
        pltpu.VMEM((RPW, H), jnp.float32),
        pltpu.SemaphoreType.DMA,
    ],
)
def _sc_gather(M_hbm, topics_hbm, out_hbm, idx_v, rows_v, sem):
    wid = lax.axis_index("s") * 2 + lax.axis_index("c")
    base = wid * RPW
    for j in range(NCH):
        pltpu.sync_copy(topics_hbm.at[pl.ds(base + j * CHUNK, CHUNK)],
                        idx_v.at[j])
    copies = [
        pltpu.async_copy(M_hbm.at[idx_v.at[j]],
                         rows_v.at[pl.ds(j * CHUNK, CHUNK)], sem)
        for j in range(NCH)
    ]
    for cp in copies:
        cp.wait()
    pltpu.sync_copy(rows_v, out_hbm.at[pl.ds(base, RPW)])


def _dec_body(h2_ref, X_ref, dg1_ref, dbe1_ref, dW2_ref, db2_ref, zl_ref,
              loss_ref, h2_scr, s2, v2, se):
    q = pl.program_id(0)
    b = pl.program_id(1)
    rows = pl.ds(b * BN, BN)

    @pl.when(q == 0)
    def _q0():
        h = h2_ref[...]
        h2_scr[rows, :] = h
        blk = jnp.sum(h, axis=0, keepdims=True)

        @pl.when(b == 0)
        def _():
            s2[...] = blk

        @pl.when(b != 0)
        def _():
            s2[...] += blk

    @pl.when(q == 1)
    def _q1():
        mu = s2[...] * INV_N
        d = h2_scr[rows, :] - mu
        blk = jnp.sum(d * d, axis=0, keepdims=True)

        @pl.when(b == 0)
        def _():
            v2[...] = blk

        @pl.when(b != 0)
        def _():
            v2[...] += blk

    @pl.when(q == 2)
    def _q2():
        mu = s2[...] * INV_N
        sd = jnp.sqrt(v2[...] * INV_N + EPS)
        t = (h2_scr[rows, :] - mu) / sd * dg1_ref[...] + dbe1_ref[...]
        r = jnp.maximum(t, 0.0)
        xr = _mm(r, dW2_ref[...]) + db2_ref[...]
        d = xr - X_ref[...]
        blk = jnp.sum(jnp.sum(d * d, axis=1, keepdims=True), axis=0,
                      keepdims=True)

        @pl.when(b == 0)
        def _():
            se[...] = blk

        @pl.when(b != 0)
        def _():
            se[...] += blk

        @pl.when(b == NB - 1)
        def _():
            loss_ref[...] = (zl_ref[...] + zl_ref[...]) + jnp.sqrt(se[...])


def kernel(X, enc_W1, enc_b1, enc_g1, enc_be1, enc_W2, enc_b2,
           dec_W1, dec_b1, dec_g1, dec_be1, dec_W2, dec_b2, codebook):
    f32 = jnp.float32
    b2row = jnp.sum(codebook * codebook, axis=1).reshape(1, K)
    cbT = codebook.T
    row = lambda v: v.reshape(1, -1)

    topics2d, zl, M = pl.pallas_call(
        _enc_body,
        grid=(3, NB),
        in_specs=[
            pl.BlockSpec((BN, D), _x_once(0)),
            pl.BlockSpec((D, H), _const),
            pl.BlockSpec((1, H), _const),
            pl.BlockSpec((1, H), _const),
            pl.BlockSpec((1, H), _const),
            pl.BlockSpec((H, C), _const),
            pl.BlockSpec((1, C), _const),
            pl.BlockSpec((C, H), _const),
            pl.BlockSpec((1, H), _const),
            pl.BlockSpec((K, C), _const),
            pl.BlockSpec((C, K), _const),
            pl.BlockSpec((1, K), _const),
        ],
        out_specs=[
            pl.BlockSpec((N, 1), _const),
            pl.BlockSpec((1, 1), _const),
            pl.BlockSpec((K, H), _const),
        ],
        out_shape=[
            jax.ShapeDtypeStruct((N, 1), jnp.int32),
            jax.ShapeDtypeStruct((1, 1), f32),
            jax.ShapeDtypeStruct((K, H), f32),
        ],
        scratch_shapes=[
            pltpu.VMEM((N, H), f32),
            pltpu.VMEM((1, H), f32),
            pltpu.VMEM((1, H), f32),
        ],
    )(X, enc_W1, row(enc_b1), row(enc_g1), row(enc_be1), enc_W2,
      row(enc_b2), dec_W1, row(dec_b1), codebook, cbT, b2row)

    h2 = _sc_gather(M, topics2d.reshape(N))

    loss2d = pl.pallas_call(
        _dec_body,
        grid=(3, NB),
        in_specs=[
            pl.BlockSpec((BN, H), _x_once(0)),
            pl.BlockSpec((BN, D), _x_once(2)),
            pl.BlockSpec((1, H), _const),
            pl.BlockSpec((1, H), _const),
            pl.BlockSpec((H, D), _const),
            pl.BlockSpec((1, D), _const),
            pl.BlockSpec((1, 1), _const),
        ],
        out_specs=pl.BlockSpec((1, 1), _const),
        out_shape=jax.ShapeDtypeStruct((1, 1), f32),
        scratch_shapes=[
            pltpu.VMEM((N, H), f32),
            pltpu.VMEM((1, H), f32),
            pltpu.VMEM((1, H), f32),
            pltpu.VMEM((1, 1), f32),
        ],
    )(h2, X, row(dec_g1), row(dec_be1), dec_W2, row(dec_b2), zl)

    return topics2d.reshape(N), loss2d[0, 0]
